# baseline (speedup 1.0000x reference)
import jax
import jax.numpy as jnp
from jax.experimental import pallas as pl
from jax.experimental.pallas import tpu as pltpu

N = 100000
D_IN = 128
D_OUT = 20
EPS = 1e-5
BLK = 5000
NB = N // BLK


def _fwd_body(x_ref, w_ref, b_ref, h_ref, s1_ref, s2_ref):
    i = pl.program_id(0)

    @pl.when(i == 0)
    def _init():
        s1_ref[...] = jnp.zeros_like(s1_ref)
        s2_ref[...] = jnp.zeros_like(s2_ref)

    z = jax.lax.dot_general(
        x_ref[...], w_ref[...],
        (((1,), (1,)), ((), ())),
        preferred_element_type=jnp.float32,
    )
    h = jnp.tanh(z + b_ref[...])
    s1_ref[...] += jnp.sum(h, axis=0, keepdims=True)
    s2_ref[...] += jnp.sum(h * h, axis=0, keepdims=True)
    h_ref[...] = h.astype(jnp.bfloat16)


def _norm_body(h_ref, s1_ref, s2_ref, g_ref, be_ref, o_ref):
    mean = s1_ref[...] * (1.0 / N)
    var = s2_ref[...] * (1.0 / N) - mean * mean
    inv = jax.lax.rsqrt(var + EPS) * g_ref[...]
    shift = be_ref[...] - mean * inv
    o_ref[...] = h_ref[...].astype(jnp.float32) * inv + shift


def kernel(x, W, b, gamma, beta):
    b2 = b.reshape(1, D_OUT)
    g2 = gamma.reshape(1, D_OUT)
    be2 = beta.reshape(1, D_OUT)
    h16, s1, s2 = pl.pallas_call(
        _fwd_body,
        grid=(NB,),
        in_specs=[
            pl.BlockSpec((BLK, D_IN), lambda i: (i, 0)),
            pl.BlockSpec((D_OUT, D_IN), lambda i: (0, 0)),
            pl.BlockSpec((1, D_OUT), lambda i: (0, 0)),
        ],
        out_specs=[
            pl.BlockSpec((BLK, D_OUT), lambda i: (i, 0)),
            pl.BlockSpec((1, D_OUT), lambda i: (0, 0)),
            pl.BlockSpec((1, D_OUT), lambda i: (0, 0)),
        ],
        out_shape=[
            jax.ShapeDtypeStruct((N, D_OUT), jnp.bfloat16),
            jax.ShapeDtypeStruct((1, D_OUT), jnp.float32),
            jax.ShapeDtypeStruct((1, D_OUT), jnp.float32),
        ],
    )(x, W, b2)
    return pl.pallas_call(
        _norm_body,
        grid=(NB,),
        in_specs=[
            pl.BlockSpec((BLK, D_OUT), lambda i: (i, 0)),
            pl.BlockSpec((1, D_OUT), lambda i: (0, 0)),
            pl.BlockSpec((1, D_OUT), lambda i: (0, 0)),
            pl.BlockSpec((1, D_OUT), lambda i: (0, 0)),
            pl.BlockSpec((1, D_OUT), lambda i: (0, 0)),
        ],
        out_specs=pl.BlockSpec((BLK, D_OUT), lambda i: (i, 0)),
        out_shape=jax.ShapeDtypeStruct((N, D_OUT), jnp.float32),
    )(h16, s1, s2, g2, be2)


# speedup vs baseline: 2.6093x; 2.6093x over previous
import jax
import jax.numpy as jnp
from jax.experimental import pallas as pl
from jax.experimental.pallas import tpu as pltpu

N = 100000
D_IN = 128
D_OUT = 20
EPS = 1e-5
BLK = 4000
NB = N // BLK


def _body(x_ref, w_ref, b_ref, g_ref, be_ref, o_ref, h_ref, s1, s2):
    i = pl.program_id(0)

    @pl.when(i == 0)
    def _init():
        s1[...] = jnp.zeros_like(s1)
        s2[...] = jnp.zeros_like(s2)

    @pl.when(i < NB)
    def _sweep1():
        z = jax.lax.dot_general(
            w_ref[...], x_ref[...],
            (((1,), (1,)), ((), ())),
            preferred_element_type=jnp.float32,
        )
        h = jnp.tanh(z + b_ref[...])
        h_ref[i] = h
        s1[...] += jnp.sum(h, axis=1, keepdims=True)
        s2[...] += jnp.sum(h * h, axis=1, keepdims=True)

    @pl.when(i == NB)
    def _finalize():
        mean = s1[...] * (1.0 / N)
        var = s2[...] * (1.0 / N) - mean * mean
        inv = jax.lax.rsqrt(var + EPS) * g_ref[...]
        shift = be_ref[...] - mean * inv
        for j in range(NB):
            o_ref[:, j * BLK:(j + 1) * BLK] = h_ref[j] * inv + shift


def kernel(x, W, b, gamma, beta):
    b2 = b.reshape(D_OUT, 1)
    g2 = gamma.reshape(D_OUT, 1)
    be2 = beta.reshape(D_OUT, 1)
    yt = pl.pallas_call(
        _body,
        grid=(NB + 1,),
        in_specs=[
            pl.BlockSpec((BLK, D_IN), lambda i: (jnp.minimum(i, NB - 1), 0)),
            pl.BlockSpec((D_OUT, D_IN), lambda i: (0, 0)),
            pl.BlockSpec((D_OUT, 1), lambda i: (0, 0)),
            pl.BlockSpec((D_OUT, 1), lambda i: (0, 0)),
            pl.BlockSpec((D_OUT, 1), lambda i: (0, 0)),
        ],
        out_specs=pl.BlockSpec((D_OUT, N), lambda i: (0, 0)),
        out_shape=jax.ShapeDtypeStruct((D_OUT, N), jnp.float32),
        scratch_shapes=[
            pltpu.VMEM((NB, D_OUT, BLK), jnp.float32),
            pltpu.VMEM((D_OUT, 1), jnp.float32),
            pltpu.VMEM((D_OUT, 1), jnp.float32),
        ],
    )(x, W, b2, g2, be2)
    return yt.T


# speedup vs baseline: 3.8818x; 1.4877x over previous
import jax
import jax.numpy as jnp
from jax.experimental import pallas as pl
from jax.experimental.pallas import tpu as pltpu

N = 100000
D_IN = 128
D_OUT = 20
EPS = 1e-5
BLK = 10000
NB = N // BLK


def _col(v_ref):
    return v_ref[...].reshape(1, D_OUT).T


def _body(x_ref, w_ref, b_ref, g_ref, be_ref, o_ref, h_ref, s1, s2):
    i = pl.program_id(0)

    @pl.when(i == 0)
    def _init():
        s1[...] = jnp.zeros_like(s1)
        s2[...] = jnp.zeros_like(s2)

    @pl.when(i < NB)
    def _sweep1():
        z = jax.lax.dot_general(
            w_ref[...], x_ref[...],
            (((1,), (1,)), ((), ())),
            preferred_element_type=jnp.float32,
        )
        h = jnp.tanh(z + _col(b_ref))
        h_ref[i] = h
        s1[...] += jnp.sum(h, axis=1, keepdims=True)
        s2[...] += jnp.sum(h * h, axis=1, keepdims=True)

    @pl.when(i == NB)
    def _finalize():
        mean = s1[...] * (1.0 / N)
        var = s2[...] * (1.0 / N) - mean * mean
        inv = jax.lax.rsqrt(var + EPS) * _col(g_ref)
        shift = _col(be_ref) - mean * inv
        for j in range(NB):
            o_ref[:, j * BLK:(j + 1) * BLK] = h_ref[j] * inv + shift


def kernel(x, W, b, gamma, beta):
    yt = pl.pallas_call(
        _body,
        grid=(NB + 1,),
        in_specs=[
            pl.BlockSpec((BLK, D_IN), lambda i: (jnp.minimum(i, NB - 1), 0)),
            pl.BlockSpec((D_OUT, D_IN), lambda i: (0, 0)),
            pl.BlockSpec((D_OUT,), lambda i: (0,)),
            pl.BlockSpec((D_OUT,), lambda i: (0,)),
            pl.BlockSpec((D_OUT,), lambda i: (0,)),
        ],
        out_specs=pl.BlockSpec((D_OUT, N), lambda i: (0, 0)),
        out_shape=jax.ShapeDtypeStruct((D_OUT, N), jnp.float32),
        scratch_shapes=[
            pltpu.VMEM((NB, D_OUT, BLK), jnp.float32),
            pltpu.VMEM((D_OUT, 1), jnp.float32),
            pltpu.VMEM((D_OUT, 1), jnp.float32),
        ],
    )(x, W, b, gamma, beta)
    return yt.T
